# Initial kernel scaffold; baseline (speedup 1.0000x reference)
import jax
import jax.numpy as jnp
from jax.experimental import pallas as pl


def kernel(edge_index, node_cat0, node_cat1, edge_cat0, edge_cat1, graph_ids, attr_embed, params):
    raise NotImplementedError("write your pallas kernel here")


# speedup vs baseline: 6.7613x; 6.7613x over previous
import functools

import jax
import jax.numpy as jnp
from jax import lax
from jax.experimental import pallas as pl
from jax.experimental.pallas import tpu as pltpu
from jax.experimental.pallas import tpu_sc as plsc

N = 10000
E = 320000
D = 128
HID = 256
G = 64
L = 5

NC = 2
NS = 16
NW = NC * NS
EPW = E // NW
CH = 80
NCHUNK = EPW // CH
RPT = 624
RPT_LAST = N - RPT * (NS - 1)
TN = 1000

_mesh = plsc.VectorSubcoreMesh(core_axis_name="c", subcore_axis_name="s")


def _tile_rows(s, fn):
    @pl.when(s < NS - 1)
    def _():
        fn(pl.multiple_of(s * RPT, 8), RPT)

    @pl.when(s == NS - 1)
    def _():
        fn((NS - 1) * RPT, RPT_LAST)


def _copy_chunk_idx(src_1d, j, dst_small):
    for i in range(CH // 16):
        dst_small[pl.ds(i * 16, 16)] = src_1d[pl.ds(j * CH + i * 16, 16)]


def _spmm_body(h_hbm, src_hbm, dst_hbm, zeros_hbm, out_hbm,
               src_v, dst_v, dch0, dch1, rows0, rows1, agg_sh, sem0, sem1):
    c = lax.axis_index("c")
    s = lax.axis_index("s")
    wid = c * NS + s
    _tile_rows(s, lambda st, sz: pltpu.sync_copy(
        zeros_hbm.at[pl.ds(st, sz)], agg_sh.at[pl.ds(st, sz)]))
    pltpu.sync_copy(src_hbm.at[pl.ds(wid * EPW, EPW)], src_v)
    pltpu.sync_copy(dst_hbm.at[pl.ds(wid * EPW, EPW)], dst_v)
    plsc.subcore_barrier()

    def gather(j, rows, sem):
        return pltpu.async_copy(h_hbm.at[src_v.at[pl.ds(j * CH, CH)]],
                                rows, sem)

    def scatter(rows, dch):
        pltpu.sync_copy(rows, agg_sh.at[dch], add=True)

    def body(i, carry):
        j = i * 2
        cp0 = gather(j, rows0, sem0)
        cp1 = gather(j + 1, rows1, sem1)
        _copy_chunk_idx(dst_v, j, dch0)
        cp0.wait()
        scatter(rows0, dch0)
        _copy_chunk_idx(dst_v, j + 1, dch1)
        cp1.wait()
        scatter(rows1, dch1)
        return carry

    lax.fori_loop(0, (NCHUNK - 1) // 2, body, 0)
    cp = gather(NCHUNK - 1, rows0, sem0)
    _copy_chunk_idx(dst_v, NCHUNK - 1, dch0)
    cp.wait()
    scatter(rows0, dch0)
    plsc.subcore_barrier()
    _tile_rows(s, lambda st, sz: pltpu.sync_copy(
        agg_sh.at[pl.ds(st, sz)], out_hbm.at[c, pl.ds(st, sz)]))


_spmm = pl.kernel(
    _spmm_body,
    out_type=jax.ShapeDtypeStruct((NC, N, D), jnp.float32),
    mesh=_mesh,
    scratch_types=[
        pltpu.VMEM((EPW,), jnp.int32),
        pltpu.VMEM((EPW,), jnp.int32),
        pltpu.VMEM((CH,), jnp.int32),
        pltpu.VMEM((CH,), jnp.int32),
        pltpu.VMEM((CH, D), jnp.float32),
        pltpu.VMEM((CH, D), jnp.float32),
        pltpu.VMEM_SHARED((N, D), jnp.float32),
        pltpu.SemaphoreType.DMA,
        pltpu.SemaphoreType.DMA,
    ],
)


def _cnt_body(oh_hbm, c0_hbm, c1_hbm, dst_hbm, zeros_hbm, out_hbm,
              c0_v, c1_v, cidx_v, dch_v, rows_v, cnt_sh, sem):
    c = lax.axis_index("c")
    s = lax.axis_index("s")
    wid = c * NS + s
    _tile_rows(s, lambda st, sz: pltpu.sync_copy(
        zeros_hbm.at[pl.ds(st, sz)], cnt_sh.at[pl.ds(st, sz)]))
    pltpu.sync_copy(c0_hbm.at[pl.ds(wid * EPW, EPW)], c0_v)
    pltpu.sync_copy(c1_hbm.at[pl.ds(wid * EPW, EPW)], c1_v)
    plsc.subcore_barrier()

    def body(j, carry):
        for i in range(CH // 16):
            a = c0_v[pl.ds(j * CH + i * 16, 16)]
            b = c1_v[pl.ds(j * CH + i * 16, 16)]
            cidx_v[pl.ds(i * 16, 16)] = a * 3 + b
        cp = pltpu.async_copy(oh_hbm.at[cidx_v], rows_v, sem)
        pltpu.sync_copy(dst_hbm.at[pl.ds(wid * EPW + j * CH, CH)], dch_v)
        cp.wait()
        pltpu.sync_copy(rows_v, cnt_sh.at[dch_v], add=True)
        return carry

    lax.fori_loop(0, NCHUNK, body, 0)
    plsc.subcore_barrier()
    _tile_rows(s, lambda st, sz: pltpu.sync_copy(
        cnt_sh.at[pl.ds(st, sz)], out_hbm.at[c, pl.ds(st, sz)]))


_cnt = pl.kernel(
    _cnt_body,
    out_type=jax.ShapeDtypeStruct((NC, N, D), jnp.float32),
    mesh=_mesh,
    scratch_types=[
        pltpu.VMEM((EPW,), jnp.int32),
        pltpu.VMEM((EPW,), jnp.int32),
        pltpu.VMEM((CH,), jnp.int32),
        pltpu.VMEM((CH,), jnp.int32),
        pltpu.VMEM((CH, D), jnp.float32),
        pltpu.VMEM_SHARED((N, D), jnp.float32),
        pltpu.SemaphoreType.DMA,
    ],
)


def _init_body(c0_ref, c1_ref, tab_ref, out_ref):
    c0 = c0_ref[...]
    c1 = c1_ref[...]
    j = lax.broadcasted_iota(jnp.int32, (TN, 128), 1)
    a = (j == c0).astype(jnp.float32) + (j == c1 + 120).astype(jnp.float32)
    out_ref[...] = jnp.dot(a, tab_ref[...], preferred_element_type=jnp.float32)


_init = pl.pallas_call(
    _init_body,
    grid=(N // TN,),
    in_specs=[
        pl.BlockSpec((TN, 1), lambda i: (i, 0)),
        pl.BlockSpec((TN, 1), lambda i: (i, 0)),
        pl.BlockSpec((128, 128), lambda i: (0, 0)),
    ],
    out_specs=pl.BlockSpec((TN, D), lambda i: (i, 0)),
    out_shape=jax.ShapeDtypeStruct((N, D), jnp.float32),
)


def _layer_body(relu, p0_ref, p1_ref, cnt_ref, t_ref, w1_ref, b1_ref,
                w2_ref, b2_ref, g_ref, bb_ref, out_ref):
    agg = p0_ref[...] + p1_ref[...] + jnp.dot(
        cnt_ref[...], t_ref[...], preferred_element_type=jnp.float32)
    z = jnp.dot(agg, w1_ref[...], preferred_element_type=jnp.float32)
    z = jnp.maximum(z + b1_ref[...], 0.0)
    z = jnp.dot(z, w2_ref[...], preferred_element_type=jnp.float32)
    z = (z + b2_ref[...]) * g_ref[...] + bb_ref[...]
    if relu:
        z = jnp.maximum(z, 0.0)
    out_ref[...] = z


def _make_layer(relu):
    return pl.pallas_call(
        functools.partial(_layer_body, relu),
        grid=(N // TN,),
        in_specs=[
            pl.BlockSpec((TN, D), lambda i: (i, 0)),
            pl.BlockSpec((TN, D), lambda i: (i, 0)),
            pl.BlockSpec((TN, D), lambda i: (i, 0)),
            pl.BlockSpec((D, D), lambda i: (0, 0)),
            pl.BlockSpec((D, HID), lambda i: (0, 0)),
            pl.BlockSpec((1, HID), lambda i: (0, 0)),
            pl.BlockSpec((HID, D), lambda i: (0, 0)),
            pl.BlockSpec((1, D), lambda i: (0, 0)),
            pl.BlockSpec((1, D), lambda i: (0, 0)),
            pl.BlockSpec((1, D), lambda i: (0, 0)),
        ],
        out_specs=pl.BlockSpec((TN, D), lambda i: (i, 0)),
        out_shape=jax.ShapeDtypeStruct((N, D), jnp.float32),
    )


_layer_mid = _make_layer(True)
_layer_last = _make_layer(False)


def _pool_body(h_ref, attr_ref, gid_ref, ww_ref, wb_ref, seg_ref, cnt_ref):
    i = pl.program_id(0)
    hf = h_ref[...] + jnp.dot(attr_ref[...], ww_ref[...],
                              preferred_element_type=jnp.float32) + wb_ref[...]
    gid = gid_ref[...]
    gi = lax.broadcasted_iota(jnp.int32, (TN, G), 1)
    a = (gi == gid).astype(jnp.float32)
    seg_p = lax.dot_general(a, hf, (((0,), (0,)), ((), ())),
                            preferred_element_type=jnp.float32)
    cnt_p = lax.dot_general(a, jnp.ones((TN, D), jnp.float32),
                            (((0,), (0,)), ((), ())),
                            preferred_element_type=jnp.float32)

    @pl.when(i == 0)
    def _():
        seg_ref[...] = jnp.zeros_like(seg_ref)
        cnt_ref[...] = jnp.zeros_like(cnt_ref)

    seg_ref[...] += seg_p
    cnt_ref[...] += cnt_p


_pool = pl.pallas_call(
    _pool_body,
    grid=(N // TN,),
    in_specs=[
        pl.BlockSpec((TN, D), lambda i: (i, 0)),
        pl.BlockSpec((TN, 256), lambda i: (i, 0)),
        pl.BlockSpec((TN, 1), lambda i: (i, 0)),
        pl.BlockSpec((256, D), lambda i: (0, 0)),
        pl.BlockSpec((1, D), lambda i: (0, 0)),
    ],
    out_specs=[
        pl.BlockSpec((G, D), lambda i: (0, 0)),
        pl.BlockSpec((G, D), lambda i: (0, 0)),
    ],
    out_shape=[
        jax.ShapeDtypeStruct((G, D), jnp.float32),
        jax.ShapeDtypeStruct((G, D), jnp.float32),
    ],
)


def _pred_body(seg_ref, cnt_ref, pw_ref, pb_ref, out_ref):
    gf = seg_ref[...] / jnp.maximum(cnt_ref[...], 1.0)
    out_ref[...] = jnp.dot(gf, pw_ref[...],
                           preferred_element_type=jnp.float32) + pb_ref[...]


_pred = pl.pallas_call(
    _pred_body,
    grid=(1,),
    in_specs=[
        pl.BlockSpec((G, D), lambda i: (0, 0)),
        pl.BlockSpec((G, D), lambda i: (0, 0)),
        pl.BlockSpec((D, 1), lambda i: (0, 0)),
        pl.BlockSpec((1, 1), lambda i: (0, 0)),
    ],
    out_specs=pl.BlockSpec((G, 1), lambda i: (0, 0)),
    out_shape=jax.ShapeDtypeStruct((G, 1), jnp.float32),
)


def kernel(edge_index, node_cat0, node_cat1, edge_cat0, edge_cat1,
           graph_ids, attr_embed, params):
    src = edge_index[0]
    dst = edge_index[1]
    c0 = edge_cat0
    c1 = edge_cat1
    zeros = jnp.zeros((N, D), jnp.float32)

    oh = jnp.zeros((18, D), jnp.float32)
    r = jnp.arange(18)
    oh = oh.at[r, r // 3].set(1.0).at[r, 6 + r % 3].set(1.0)

    ntab = jnp.concatenate([
        params['node_emb0'], params['node_emb1'],
        jnp.zeros((5, D), jnp.float32)], axis=0)

    cnt2 = _cnt(oh, c0, c1, dst, zeros)
    cnt = cnt2[0] + cnt2[1]

    h = _init(node_cat0.reshape(N, 1), node_cat1.reshape(N, 1), ntab)

    for l in range(L):
        p = params['layers'][l]
        t = jnp.concatenate([
            p['edge_emb0'], p['edge_emb1'],
            jnp.zeros((D - 9, D), jnp.float32)], axis=0)
        parts = _spmm(h, src, dst, zeros)
        layer_fn = _layer_mid if l < L - 1 else _layer_last
        h = layer_fn(parts[0], parts[1], cnt, t,
                     p['W1'], p['b1'].reshape(1, HID),
                     p['W2'], p['b2'].reshape(1, D),
                     p['bn_g'].reshape(1, D), p['bn_b'].reshape(1, D))

    seg, cntg = _pool(h, attr_embed, graph_ids.reshape(N, 1),
                      params['w_W'], params['w_b'].reshape(1, D))
    return _pred(seg, cntg, params['pred_W'],
                 params['pred_b'].reshape(1, 1))


# speedup vs baseline: 7.4081x; 1.0957x over previous
import functools

import jax
import jax.numpy as jnp
from jax import lax
from jax.experimental import pallas as pl
from jax.experimental.pallas import tpu as pltpu
from jax.experimental.pallas import tpu_sc as plsc

N = 10000
E = 320000
D = 128
HID = 256
G = 64
L = 5

NC = 2
NS = 16
NW = NC * NS
EPW = E // NW
CH = 96
NCHUNK = -(-EPW // CH)
EPW_P = NCHUNK * CH
NP = N + 16
RPT = 624
RPT_LAST = N - RPT * (NS - 1)
TN = 1000

_mesh = plsc.VectorSubcoreMesh(core_axis_name="c", subcore_axis_name="s")


def _tile_rows(s, fn):
    @pl.when(s < NS - 1)
    def _():
        fn(pl.multiple_of(s * RPT, 8), RPT)

    @pl.when(s == NS - 1)
    def _():
        fn((NS - 1) * RPT, RPT_LAST)


def _copy_chunk_idx(src_1d, j, dst_small):
    for i in range(CH // 16):
        dst_small[pl.ds(i * 16, 16)] = src_1d[pl.ds(j * CH + i * 16, 16)]


def _spmm_body(h_hbm, src_hbm, dst_hbm, zeros_hbm, out_hbm,
               src_v, dst_v, dch0, dch1, rows0, rows1, agg_sh, sem0, sem1):
    c = lax.axis_index("c")
    s = lax.axis_index("s")
    wid = c * NS + s
    _tile_rows(s, lambda st, sz: pltpu.sync_copy(
        zeros_hbm.at[pl.ds(st, sz)], agg_sh.at[pl.ds(st, sz)]))
    pltpu.sync_copy(src_hbm.at[pl.ds(wid * EPW_P, EPW_P)], src_v)
    pltpu.sync_copy(dst_hbm.at[pl.ds(wid * EPW_P, EPW_P)], dst_v)
    plsc.subcore_barrier()

    def gather(j, rows, sem):
        return pltpu.async_copy(h_hbm.at[src_v.at[pl.ds(j * CH, CH)]],
                                rows, sem)

    def gwait(rows, sem):
        pltpu.make_async_copy(h_hbm.at[src_v.at[pl.ds(0, CH)]], rows,
                              sem).wait()

    def scatter(rows, dch):
        pltpu.sync_copy(rows, agg_sh.at[dch], add=True)

    gather(0, rows0, sem0)

    def body(p, carry):
        j = p * 2
        gwait(rows0, sem0)
        gather(j + 1, rows1, sem1)
        _copy_chunk_idx(dst_v, j, dch0)
        scatter(rows0, dch0)
        gwait(rows1, sem1)
        gather(j + 2, rows0, sem0)
        _copy_chunk_idx(dst_v, j + 1, dch1)
        scatter(rows1, dch1)
        return carry

    lax.fori_loop(0, (NCHUNK - 1) // 2, body, 0)
    gwait(rows0, sem0)
    _copy_chunk_idx(dst_v, NCHUNK - 1, dch0)
    scatter(rows0, dch0)
    plsc.subcore_barrier()
    _tile_rows(s, lambda st, sz: pltpu.sync_copy(
        agg_sh.at[pl.ds(st, sz)], out_hbm.at[c, pl.ds(st, sz)]))


_spmm = pl.kernel(
    _spmm_body,
    out_type=jax.ShapeDtypeStruct((NC, N, D), jnp.float32),
    mesh=_mesh,
    scratch_types=[
        pltpu.VMEM((EPW_P,), jnp.int32),
        pltpu.VMEM((EPW_P,), jnp.int32),
        pltpu.VMEM((CH,), jnp.int32),
        pltpu.VMEM((CH,), jnp.int32),
        pltpu.VMEM((CH, D), jnp.float32),
        pltpu.VMEM((CH, D), jnp.float32),
        pltpu.VMEM_SHARED((NP, D), jnp.float32),
        pltpu.SemaphoreType.DMA,
        pltpu.SemaphoreType.DMA,
    ],
)


KREP = 512
EROWS = (NW * EPW_P) // 128


def _cidx_body(c0_ref, c1_ref, out_ref):
    pos = (lax.broadcasted_iota(jnp.int32, (EROWS, 128), 0) * 128
           + lax.broadcasted_iota(jnp.int32, (EROWS, 128), 1))
    out_ref[...] = (c0_ref[...] * 3 + c1_ref[...]
                    + 18 * (pos & (KREP - 1)))


_cidx = pl.pallas_call(
    _cidx_body,
    grid=(1,),
    in_specs=[
        pl.BlockSpec((EROWS, 128), lambda i: (0, 0)),
        pl.BlockSpec((EROWS, 128), lambda i: (0, 0)),
    ],
    out_specs=pl.BlockSpec((EROWS, 128), lambda i: (0, 0)),
    out_shape=jax.ShapeDtypeStruct((EROWS, 128), jnp.int32),
)


def _init_body(c0_ref, c1_ref, tab_ref, out_ref):
    c0 = c0_ref[...]
    c1 = c1_ref[...]
    j = lax.broadcasted_iota(jnp.int32, (TN, 128), 1)
    a = (j == c0).astype(jnp.float32) + (j == c1 + 120).astype(jnp.float32)
    out_ref[...] = jnp.dot(a, tab_ref[...], preferred_element_type=jnp.float32)


_init = pl.pallas_call(
    _init_body,
    grid=(N // TN,),
    in_specs=[
        pl.BlockSpec((TN, 1), lambda i: (i, 0)),
        pl.BlockSpec((TN, 1), lambda i: (i, 0)),
        pl.BlockSpec((128, 128), lambda i: (0, 0)),
    ],
    out_specs=pl.BlockSpec((TN, D), lambda i: (i, 0)),
    out_shape=jax.ShapeDtypeStruct((N, D), jnp.float32),
)


def _layer_body(relu, p0_ref, p1_ref, cnt_ref, t_ref, w1_ref, b1_ref,
                w2_ref, b2_ref, g_ref, bb_ref, out_ref):
    agg = p0_ref[...] + p1_ref[...] + jnp.dot(
        cnt_ref[...], t_ref[...], preferred_element_type=jnp.float32)
    z = jnp.dot(agg, w1_ref[...], preferred_element_type=jnp.float32)
    z = jnp.maximum(z + b1_ref[...], 0.0)
    z = jnp.dot(z, w2_ref[...], preferred_element_type=jnp.float32)
    z = (z + b2_ref[...]) * g_ref[...] + bb_ref[...]
    if relu:
        z = jnp.maximum(z, 0.0)
    out_ref[...] = z


def _make_layer(relu):
    return pl.pallas_call(
        functools.partial(_layer_body, relu),
        grid=(N // TN,),
        in_specs=[
            pl.BlockSpec((TN, D), lambda i: (i, 0)),
            pl.BlockSpec((TN, D), lambda i: (i, 0)),
            pl.BlockSpec((TN, D), lambda i: (i, 0)),
            pl.BlockSpec((D, D), lambda i: (0, 0)),
            pl.BlockSpec((D, HID), lambda i: (0, 0)),
            pl.BlockSpec((1, HID), lambda i: (0, 0)),
            pl.BlockSpec((HID, D), lambda i: (0, 0)),
            pl.BlockSpec((1, D), lambda i: (0, 0)),
            pl.BlockSpec((1, D), lambda i: (0, 0)),
            pl.BlockSpec((1, D), lambda i: (0, 0)),
        ],
        out_specs=pl.BlockSpec((TN, D), lambda i: (i, 0)),
        out_shape=jax.ShapeDtypeStruct((N, D), jnp.float32),
    )


_layer_mid = _make_layer(True)
_layer_last = _make_layer(False)


def _pool_body(h_ref, attr_ref, gid_ref, ww_ref, wb_ref, seg_ref, cnt_ref):
    i = pl.program_id(0)
    hf = h_ref[...] + jnp.dot(attr_ref[...], ww_ref[...],
                              preferred_element_type=jnp.float32) + wb_ref[...]
    gid = gid_ref[...]
    gi = lax.broadcasted_iota(jnp.int32, (TN, G), 1)
    a = (gi == gid).astype(jnp.float32)
    seg_p = lax.dot_general(a, hf, (((0,), (0,)), ((), ())),
                            preferred_element_type=jnp.float32)
    cnt_p = lax.dot_general(a, jnp.ones((TN, D), jnp.float32),
                            (((0,), (0,)), ((), ())),
                            preferred_element_type=jnp.float32)

    @pl.when(i == 0)
    def _():
        seg_ref[...] = jnp.zeros_like(seg_ref)
        cnt_ref[...] = jnp.zeros_like(cnt_ref)

    seg_ref[...] += seg_p
    cnt_ref[...] += cnt_p


_pool = pl.pallas_call(
    _pool_body,
    grid=(N // TN,),
    in_specs=[
        pl.BlockSpec((TN, D), lambda i: (i, 0)),
        pl.BlockSpec((TN, 256), lambda i: (i, 0)),
        pl.BlockSpec((TN, 1), lambda i: (i, 0)),
        pl.BlockSpec((256, D), lambda i: (0, 0)),
        pl.BlockSpec((1, D), lambda i: (0, 0)),
    ],
    out_specs=[
        pl.BlockSpec((G, D), lambda i: (0, 0)),
        pl.BlockSpec((G, D), lambda i: (0, 0)),
    ],
    out_shape=[
        jax.ShapeDtypeStruct((G, D), jnp.float32),
        jax.ShapeDtypeStruct((G, D), jnp.float32),
    ],
)


def _pred_body(seg_ref, cnt_ref, pw_ref, pb_ref, out_ref):
    gf = seg_ref[...] / jnp.maximum(cnt_ref[...], 1.0)
    out_ref[...] = jnp.dot(gf, pw_ref[...],
                           preferred_element_type=jnp.float32) + pb_ref[...]


_pred = pl.pallas_call(
    _pred_body,
    grid=(1,),
    in_specs=[
        pl.BlockSpec((G, D), lambda i: (0, 0)),
        pl.BlockSpec((G, D), lambda i: (0, 0)),
        pl.BlockSpec((D, 1), lambda i: (0, 0)),
        pl.BlockSpec((1, 1), lambda i: (0, 0)),
    ],
    out_specs=pl.BlockSpec((G, 1), lambda i: (0, 0)),
    out_shape=jax.ShapeDtypeStruct((G, 1), jnp.float32),
)


def kernel(edge_index, node_cat0, node_cat1, edge_cat0, edge_cat1,
           graph_ids, attr_embed, params):
    def pad_pw(x, fill):
        return jnp.pad(x.reshape(NW, EPW), ((0, 0), (0, EPW_P - EPW)),
                       constant_values=fill).reshape(-1)

    src = pad_pw(edge_index[0], 0)
    dst = pad_pw(edge_index[1], N)
    c0 = pad_pw(edge_cat0, 0)
    c1 = pad_pw(edge_cat1, 0)
    zeros = jnp.zeros((N, D), jnp.float32)

    r = jnp.arange(18 * KREP)
    ohrep = (jnp.zeros((N, D), jnp.float32)
             .at[r, (r % 18) // 3].set(1.0)
             .at[r, 6 + (r % 18) % 3].set(1.0))

    ntab = jnp.concatenate([
        params['node_emb0'], params['node_emb1'],
        jnp.zeros((5, D), jnp.float32)], axis=0)

    cidx = _cidx(c0.reshape(EROWS, 128), c1.reshape(EROWS, 128)).reshape(-1)
    cnt2 = _spmm(ohrep, cidx, dst, zeros)
    cnt = cnt2[0] + cnt2[1]

    h = _init(node_cat0.reshape(N, 1), node_cat1.reshape(N, 1), ntab)

    for l in range(L):
        p = params['layers'][l]
        t = jnp.concatenate([
            p['edge_emb0'], p['edge_emb1'],
            jnp.zeros((D - 9, D), jnp.float32)], axis=0)
        parts = _spmm(h, src, dst, zeros)
        layer_fn = _layer_mid if l < L - 1 else _layer_last
        h = layer_fn(parts[0], parts[1], cnt, t,
                     p['W1'], p['b1'].reshape(1, HID),
                     p['W2'], p['b2'].reshape(1, D),
                     p['bn_g'].reshape(1, D), p['bn_b'].reshape(1, D))

    seg, cntg = _pool(h, attr_embed, graph_ids.reshape(N, 1),
                      params['w_W'], params['w_b'].reshape(1, D))
    return _pred(seg, cntg, params['pred_W'],
                 params['pred_b'].reshape(1, 1))
